# baseline (speedup 1.0000x reference)
import functools

import numpy as np
import jax
import jax.numpy as jnp
from jax import lax
from jax.experimental import pallas as pl
from jax.experimental.pallas import tpu as pltpu
from jax.experimental.pallas import tpu_sc as plsc

M = 64
Q = M * M
B_TOTAL = 16384
BB = 1024

NC = 2
NS = 16
NW = NC * NS
ROWS_PER_W = B_TOTAL // NW
IDX_CHUNK = 128
CHUNKS = ROWS_PER_W // IDX_CHUNK

_W_CONST = (1.0 / (np.arange(Q, dtype=np.float64) + 2.0)).astype(
    np.float32).reshape(1, Q)
_W1D_CONST = _W_CONST.reshape(Q)
_D_CONST = np.tile(np.eye(M, dtype=np.float32), (1, M))
_E_CONST = np.repeat(np.eye(M, dtype=np.float32), M, axis=0)

_S1 = np.float32(-6.28128131)
_S3 = np.float32(41.09534543)
_S5 = np.float32(-73.5871216)


B_SC = 2048
B_TC = B_TOTAL - B_SC
SC_ROWS_W = B_SC // NW
W_START = B_TC // ROWS_PER_W


def _sc_gather(tok1d, emb):
    mesh = plsc.VectorSubcoreMesh(core_axis_name="c", subcore_axis_name="s")

    @functools.partial(
        pl.kernel,
        mesh=mesh,
        out_type=(
            jax.ShapeDtypeStruct((B_TOTAL, M), jnp.float32),
            jax.ShapeDtypeStruct((B_SC, M), jnp.float32),
        ),
        scratch_types=[
            pltpu.VMEM((ROWS_PER_W,), jnp.int32),
            pltpu.VMEM((ROWS_PER_W, M), jnp.float32),
            pltpu.SemaphoreType.DMA,
        ],
        compiler_params=pltpu.CompilerParams(use_tc_tiling_on_sc=False),
    )
    def gk(tok_hbm, emb_hbm, out_hbm, xsc_hbm, idx_v, rows_v, sem):
        wid = lax.axis_index("s") * NC + lax.axis_index("c")
        pltpu.sync_copy(tok_hbm.at[pl.ds(wid * ROWS_PER_W, ROWS_PER_W)], idx_v)
        copies = [
            pltpu.async_copy(
                emb_hbm.at[idx_v.at[pl.ds(j * IDX_CHUNK, IDX_CHUNK)]],
                rows_v.at[pl.ds(j * IDX_CHUNK, IDX_CHUNK)],
                sem,
            )
            for j in range(CHUNKS)
        ]
        for c in copies:
            c.wait()
        pltpu.sync_copy(rows_v, out_hbm.at[pl.ds(wid * ROWS_PER_W, ROWS_PER_W)])

        @pl.when(wid >= W_START)
        def _():
            pltpu.sync_copy(
                rows_v,
                xsc_hbm.at[pl.ds((wid - W_START) * ROWS_PER_W, ROWS_PER_W)])

    return gk(tok1d, emb)


def _sc_dense(k1d, xsc, pflat, invp):
    mesh = plsc.VectorSubcoreMesh(core_axis_name="c", subcore_axis_name="s")

    @functools.partial(
        pl.kernel,
        mesh=mesh,
        out_type=jax.ShapeDtypeStruct((B_SC, M), jnp.float32),
        scratch_types=[
            pltpu.VMEM((SC_ROWS_W,), jnp.float32),
            pltpu.VMEM((SC_ROWS_W, M), jnp.float32),
            pltpu.VMEM((Q,), jnp.float32),
            pltpu.VMEM((Q,), jnp.float32),
            pltpu.VMEM((SC_ROWS_W, M), jnp.float32),
        ],
        compiler_params=pltpu.CompilerParams(
            use_tc_tiling_on_sc=False, needs_layout_passes=False),
    )
    def dk(k_hbm, x_hbm, p_hbm, invp_hbm, out_hbm, k_v, x_v, invp_v, p_v,
           out_v):
        wid = lax.axis_index("s") * NC + lax.axis_index("c")
        base = wid * SC_ROWS_W
        pltpu.sync_copy(k_hbm.at[pl.ds(B_TC + base, SC_ROWS_W)], k_v)
        pltpu.sync_copy(x_hbm.at[pl.ds(base, SC_ROWS_W)], x_v)
        pltpu.sync_copy(invp_hbm, invp_v)
        pltpu.sync_copy(p_hbm, p_v)
        lane = lax.iota(jnp.int32, 16)

        def row(b, _):
            kv = plsc.load_gather(k_v, [jnp.full((16,), b, jnp.int32)])

            def col(i, _):
                acc = jnp.zeros((16,), jnp.float32)
                for v in range(M // 16):
                    q0 = i * M + v * 16
                    inv = invp_v[pl.ds(q0, 16)]
                    pv = p_v[pl.ds(q0, 16)]
                    xv = plsc.load_gather(
                        x_v, [jnp.full((16,), b, jnp.int32),
                              lane + (v * 16)])
                    f = kv * inv
                    fh = f + 0.5
                    r = f - fh.astype(jnp.int32).astype(jnp.float32)
                    w = jnp.abs(r) - 0.25
                    w2 = w * w
                    g = w * (_S1 + w2 * (_S3 + w2 * _S5))
                    acc = acc + g * (xv * pv)
                s = jnp.sum(acc)
                plsc.store_scatter(
                    out_v,
                    [jnp.full((16,), b, jnp.int32),
                     jnp.full((16,), i, jnp.int32)],
                    jnp.full((16,), s, jnp.float32),
                    mask=(lane == 0))
                return 0

            lax.fori_loop(0, M, col, 0, unroll=8)
            return 0

        lax.fori_loop(0, SC_ROWS_W, row, 0)
        pltpu.sync_copy(out_v, out_hbm.at[pl.ds(base, SC_ROWS_W)])

    return dk(k1d, xsc, pflat, invp)


def _dd_body(k_ref, x_ref, d2_ref, e_ref, w_ref, o_ref):
    f = k_ref[...] * w_ref[...]
    w = jnp.abs(f - lax.round(f, lax.RoundingMethod.TO_NEAREST_EVEN)) - 0.25
    w2 = w * w
    g = w * (_S1 + w2 * (_S3 + w2 * _S5))
    xt = jnp.dot(x_ref[...].astype(jnp.bfloat16), d2_ref[...],
                 preferred_element_type=jnp.float32)
    o_ref[...] = jnp.dot((g * xt).astype(jnp.bfloat16), e_ref[...],
                         preferred_element_type=jnp.float32)


def _dense_tc(k2d, x, d2, n_rows):
    grid = (n_rows // BB,)
    return pl.pallas_call(
        _dd_body,
        grid=grid,
        in_specs=[
            pl.BlockSpec((BB, 1), lambda i: (i, 0)),
            pl.BlockSpec((BB, M), lambda i: (i, 0)),
            pl.BlockSpec((M, Q), lambda i: (0, 0)),
            pl.BlockSpec((Q, M), lambda i: (0, 0)),
            pl.BlockSpec((1, Q), lambda i: (0, 0)),
        ],
        out_specs=pl.BlockSpec((BB, M), lambda i: (i, 0)),
        out_shape=jax.ShapeDtypeStruct((n_rows, M), jnp.float32),
        compiler_params=pltpu.CompilerParams(
            dimension_semantics=("parallel",)),
    )(k2d, x, d2, jnp.asarray(_E_CONST).astype(jnp.bfloat16),
      jnp.asarray(_W_CONST))


def kernel(k_tensor, token_indices, emb, P):
    x, xsc = _sc_gather(token_indices.astype(jnp.int32), emb)
    d2 = (jnp.asarray(_D_CONST) * P.reshape(1, Q)).astype(jnp.bfloat16)
    nk_tc = _dense_tc(k_tensor.reshape(-1, 1), x, d2, B_TC)
    nk_sc = _sc_dense(k_tensor, xsc, P.reshape(Q), jnp.asarray(_W1D_CONST))
    return jnp.concatenate([nk_tc, nk_sc], axis=0)


# speedup vs baseline: 1.1738x; 1.1738x over previous
import functools

import numpy as np
import jax
import jax.numpy as jnp
from jax import lax
from jax.experimental import pallas as pl
from jax.experimental.pallas import tpu as pltpu
from jax.experimental.pallas import tpu_sc as plsc

M = 64
Q = M * M
B_TOTAL = 16384
BB = 2048

NC = 2
NS = 16
NW = NC * NS
ROWS_PER_W = B_TOTAL // NW
IDX_CHUNK = 128
CHUNKS = ROWS_PER_W // IDX_CHUNK

_W_CONST = (1.0 / (np.arange(Q, dtype=np.float64) + 2.0)).astype(
    np.float32).reshape(1, Q)
_D_CONST = np.tile(np.eye(M, dtype=np.float32), (1, M))
_E_CONST = np.repeat(np.eye(M, dtype=np.float32), M, axis=0)

_S1 = np.float32(-6.28128131)
_S3 = np.float32(41.09534543)
_S5 = np.float32(-73.5871216)


def _sc_gather(tok1d, emb):
    mesh = plsc.VectorSubcoreMesh(core_axis_name="c", subcore_axis_name="s")

    @functools.partial(
        pl.kernel,
        mesh=mesh,
        out_type=jax.ShapeDtypeStruct((B_TOTAL, M), jnp.float32),
        scratch_types=[
            pltpu.VMEM((ROWS_PER_W,), jnp.int32),
            pltpu.VMEM((ROWS_PER_W, M), jnp.float32),
            pltpu.SemaphoreType.DMA,
        ],
        compiler_params=pltpu.CompilerParams(use_tc_tiling_on_sc=False),
    )
    def gk(tok_hbm, emb_hbm, out_hbm, idx_v, rows_v, sem):
        wid = lax.axis_index("s") * NC + lax.axis_index("c")
        pltpu.sync_copy(tok_hbm.at[pl.ds(wid * ROWS_PER_W, ROWS_PER_W)], idx_v)
        copies = [
            pltpu.async_copy(
                emb_hbm.at[idx_v.at[pl.ds(j * IDX_CHUNK, IDX_CHUNK)]],
                rows_v.at[pl.ds(j * IDX_CHUNK, IDX_CHUNK)],
                sem,
            )
            for j in range(CHUNKS)
        ]
        for c in copies:
            c.wait()
        pltpu.sync_copy(rows_v, out_hbm.at[pl.ds(wid * ROWS_PER_W, ROWS_PER_W)])

    return gk(tok1d, emb)


def _dd_body(k_ref, x_ref, d2_ref, e_ref, w_ref, o_ref):
    f = k_ref[...] * w_ref[...]
    w = jnp.abs(f - lax.round(f, lax.RoundingMethod.TO_NEAREST_EVEN)) - 0.25
    w2 = w * w
    g = w * (_S1 + w2 * (_S3 + w2 * _S5))
    xt = jnp.dot(x_ref[...].astype(jnp.bfloat16), d2_ref[...],
                 preferred_element_type=jnp.float32)
    o_ref[...] = jnp.dot((g * xt).astype(jnp.bfloat16), e_ref[...],
                         preferred_element_type=jnp.float32)


def _dense_tc(k2d, x, d2):
    grid = (k2d.shape[0] // BB,)
    return pl.pallas_call(
        _dd_body,
        grid=grid,
        in_specs=[
            pl.BlockSpec((BB, 1), lambda i: (i, 0)),
            pl.BlockSpec((BB, M), lambda i: (i, 0)),
            pl.BlockSpec((M, Q), lambda i: (0, 0)),
            pl.BlockSpec((Q, M), lambda i: (0, 0)),
            pl.BlockSpec((1, Q), lambda i: (0, 0)),
        ],
        out_specs=pl.BlockSpec((BB, M), lambda i: (i, 0)),
        out_shape=jax.ShapeDtypeStruct((k2d.shape[0], M), jnp.float32),
        compiler_params=pltpu.CompilerParams(
            dimension_semantics=("parallel",)),
    )(k2d, x, d2, jnp.asarray(_E_CONST).astype(jnp.bfloat16),
      jnp.asarray(_W_CONST))


def kernel(k_tensor, token_indices, emb, P):
    x = _sc_gather(token_indices.astype(jnp.int32), emb)
    d2 = (jnp.asarray(_D_CONST) * P.reshape(1, Q)).astype(jnp.bfloat16)
    return _dense_tc(k_tensor.reshape(-1, 1), x, d2)
